# baseline (speedup 1.0000x reference)
import functools

import jax
import jax.numpy as jnp
from jax import lax
from jax.experimental import pallas as pl
from jax.experimental.pallas import tpu as pltpu
from jax.experimental.pallas import tpu_sc as plsc

_HIDDEN = 2048
_EXPERTS = 16
_TOKENS = 16384
_THRESH = 0.2
_BM = 1024

_info = plsc.get_sparse_core_info()
_NC, _NSUB, _L = _info.num_cores, _info.num_subcores, _info.num_lanes
_NW = _NC * _NSUB
_CHUNK = _TOKENS // _NW


def _logits_block(h_ref, w_ref, b_ref, out_ref):
    out_ref[...] = lax.dot_general(
        w_ref[...], h_ref[...], (((1,), (1,)), ((), ())),
        preferred_element_type=jnp.float32,
    ) + b_ref[...]


def _tc_logits(hidden_states, W, b2):
    grid = (_TOKENS // _BM,)
    return pl.pallas_call(
        _logits_block,
        grid=grid,
        in_specs=[
            pl.BlockSpec((_BM, _HIDDEN), lambda i: (i, 0)),
            pl.BlockSpec((_EXPERTS, _HIDDEN), lambda i: (0, 0)),
            pl.BlockSpec((_EXPERTS, 1), lambda i: (0, 0)),
        ],
        out_specs=pl.BlockSpec((_EXPERTS, _BM), lambda i: (0, i)),
        out_shape=jax.ShapeDtypeStruct((_EXPERTS, _TOKENS), jnp.float32),
        compiler_params=pltpu.CompilerParams(
            dimension_semantics=("parallel",),
        ),
    )(hidden_states, W, b2)


@functools.partial(
    pl.kernel,
    mesh=plsc.VectorSubcoreMesh(core_axis_name="c", subcore_axis_name="s"),
    out_type=(
        jax.ShapeDtypeStruct((2, _TOKENS), jnp.int32),
        jax.ShapeDtypeStruct((2, _TOKENS), jnp.float32),
    ),
    scratch_types=[
        pltpu.VMEM((_EXPERTS, _CHUNK), jnp.float32),
        pltpu.VMEM((2, _CHUNK), jnp.int32),
        pltpu.VMEM((2, _CHUNK), jnp.float32),
    ],
)
def _sc_top2(logits_hbm, idx_hbm, mask_hbm, l_v, idx_v, mask_v):
    wid = lax.axis_index("s") * _NC + lax.axis_index("c")
    base = wid * _CHUNK
    pltpu.sync_copy(logits_hbm.at[:, pl.ds(base, _CHUNK)], l_v)

    def group(g, carry):
        off = g * _L
        m1 = l_v[0, pl.ds(off, _L)]
        i1 = jnp.zeros((_L,), jnp.int32)
        m2 = jnp.full((_L,), -jnp.inf, jnp.float32)
        i2 = jnp.zeros((_L,), jnp.int32)
        for e in range(1, _EXPERTS):
            le = l_v[e, pl.ds(off, _L)]
            ev = jnp.full((_L,), e, jnp.int32)
            is1 = le > m1
            is2 = le > m2
            m2n = jnp.where(is2, le, m2)
            i2n = jnp.where(is2, ev, i2)
            m2 = jnp.where(is1, m1, m2n)
            i2 = jnp.where(is1, i1, i2n)
            m1 = jnp.where(is1, le, m1)
            i1 = jnp.where(is1, ev, i1)
        idx_v[0, pl.ds(off, _L)] = i1
        idx_v[1, pl.ds(off, _L)] = i2
        one = jnp.ones((_L,), jnp.float32)
        zero = jnp.zeros((_L,), jnp.float32)
        mask_v[0, pl.ds(off, _L)] = jnp.where(m1 > _THRESH, one, zero)
        mask_v[1, pl.ds(off, _L)] = jnp.where(m2 > _THRESH, one, zero)
        return carry

    lax.fori_loop(0, _CHUNK // _L, group, 0)
    pltpu.sync_copy(idx_v, idx_hbm.at[:, pl.ds(base, _CHUNK)])
    pltpu.sync_copy(mask_v, mask_hbm.at[:, pl.ds(base, _CHUNK)])


def kernel(hidden_states, W, b):
    b2 = b.reshape(_EXPERTS, 1)
    logits_t = _tc_logits(hidden_states, W, b2)
    idx_t, mask_t = _sc_top2(logits_t)
    return (idx_t.T, mask_t.T)


# speedup vs baseline: 1.4223x; 1.4223x over previous
import jax
import jax.numpy as jnp
from jax import lax
from jax.experimental import pallas as pl
from jax.experimental.pallas import tpu as pltpu

_HIDDEN = 2048
_EXPERTS = 16
_THRESH = 0.2
_BM = 1024


def _router_block(h_ref, w_ref, b_ref, idx_ref, mask_ref):
    logits = lax.dot_general(
        w_ref[...], h_ref[...], (((1,), (1,)), ((), ())),
        preferred_element_type=jnp.float32,
    ) + b_ref[...]
    bm = logits.shape[1]
    iota = lax.broadcasted_iota(jnp.int32, (_EXPERTS, bm), 0)
    m1 = jnp.max(logits, axis=0, keepdims=True)
    i1 = jnp.min(jnp.where(logits == m1, iota, _EXPERTS), axis=0, keepdims=True)
    masked = jnp.where(iota == i1, -jnp.inf, logits)
    m2 = jnp.max(masked, axis=0, keepdims=True)
    i2 = jnp.min(jnp.where(masked == m2, iota, _EXPERTS), axis=0, keepdims=True)
    idx_ref[...] = jnp.concatenate([i1, i2], axis=0)
    mask_ref[...] = (jnp.concatenate([m1, m2], axis=0) > _THRESH).astype(jnp.float32)


def kernel(hidden_states, W, b):
    tokens = hidden_states.shape[0]
    grid = (tokens // _BM,)
    b2 = b.reshape(_EXPERTS, 1)
    out_shapes = (
        jax.ShapeDtypeStruct((2, tokens), jnp.int32),
        jax.ShapeDtypeStruct((2, tokens), jnp.float32),
    )
    idx_t, mask_t = pl.pallas_call(
        _router_block,
        grid=grid,
        in_specs=[
            pl.BlockSpec((_BM, _HIDDEN), lambda i: (i, 0)),
            pl.BlockSpec((_EXPERTS, _HIDDEN), lambda i: (0, 0)),
            pl.BlockSpec((_EXPERTS, 1), lambda i: (0, 0)),
        ],
        out_specs=(
            pl.BlockSpec((2, _BM), lambda i: (0, i)),
            pl.BlockSpec((2, _BM), lambda i: (0, i)),
        ),
        out_shape=out_shapes,
        compiler_params=pltpu.CompilerParams(
            dimension_semantics=("parallel",),
        ),
    )(hidden_states, W, b2)
    return (idx_t.T, mask_t.T)
